# baseline (speedup 1.0000x reference)
import functools
import math

import jax
import jax.numpy as jnp
from jax import lax
from jax.experimental import pallas as pl
from jax.experimental.pallas import tpu as pltpu
import jax.experimental.pallas.tpu_sc as plsc

N_NODES = 10000
FEAT = 128
NP = 10240
NC = 2
NS = 16
NW = NC * NS
CHUNK = 128
SCH = 16
DSCH = 40
RPT = NP // NS
RB = 1024
SKEW = (10, 0)

_mesh = plsc.VectorSubcoreMesh(core_axis_name="c", subcore_axis_name="s")


def _deg_body(dst2d, ones_hbm, zeros_hbm, out, didx_v, ones_v, acc_sh, sem):
    cid = lax.axis_index("c")
    sid = lax.axis_index("s")
    wid = sid * NC + cid
    rows_pw = dst2d.shape[0] // NW
    base = wid * rows_pw

    pltpu.sync_copy(zeros_hbm, acc_sh.at[pl.ds(sid * RPT, RPT)])
    pltpu.sync_copy(ones_hbm, ones_v)
    plsc.subcore_barrier()

    for s in range(rows_pw // DSCH):
        pltpu.sync_copy(dst2d.at[pl.ds(base + s * DSCH, DSCH)], didx_v)
        for c0 in range(0, DSCH, 8):
            descs = [
                pltpu.async_copy(ones_v, acc_sh.at[didx_v.at[c0 + t]],
                                 sem, add=True)
                for t in range(8)
            ]
            for d in descs:
                d.wait()

    plsc.subcore_barrier()
    pltpu.sync_copy(acc_sh.at[pl.ds(sid * RPT, RPT)],
                    out.at[cid, pl.ds(sid * RPT, RPT)])


def _agg_body(g_hbm, src2d, dst2d, zeros_hbm, out,
              sidx_v, didx_v, rows_v, acc_sh, gs0, gs1):
    cid = lax.axis_index("c")
    sid = lax.axis_index("s")
    gsem = (gs0, gs1)

    pltpu.sync_copy(zeros_hbm, acc_sh.at[pl.ds(sid * RPT, RPT)])
    plsc.subcore_barrier()

    total_sch = src2d.shape[0] // SCH
    per_pair = total_sch // NS
    nsc0 = per_pair * SKEW[0] // (SKEW[0] + SKEW[1])
    nsc1 = per_pair - nsc0
    rows0 = NS * nsc0 * SCH

    def run(base_row, nsc):
        for s in range(nsc):
            pltpu.sync_copy(src2d.at[pl.ds(base_row + s * SCH, SCH)], sidx_v)
            pltpu.sync_copy(dst2d.at[pl.ds(base_row + s * SCH, SCH)], didx_v)
            pend = [
                pltpu.async_copy(g_hbm.at[sidx_v.at[b]], rows_v.at[b],
                                 gsem[b])
                for b in range(2)
            ]
            for c in range(SCH):
                b = c % 2
                pend[b].wait()
                pltpu.sync_copy(rows_v.at[b], acc_sh.at[didx_v.at[c]],
                                add=True)
                if c + 2 < SCH:
                    pend[b] = pltpu.async_copy(
                        g_hbm.at[sidx_v.at[c + 2]], rows_v.at[b], gsem[b])

    @pl.when(cid == 0)
    def _():
        run(sid * nsc0 * SCH, nsc0)

    @pl.when(cid == 1)
    def _():
        run(rows0 + sid * nsc1 * SCH, nsc1)

    plsc.subcore_barrier()
    pltpu.sync_copy(acc_sh.at[pl.ds(sid * RPT, RPT)],
                    out.at[cid, pl.ds(sid * RPT, RPT)])


def _make_deg_kernel():
    return pl.kernel(
        _deg_body,
        out_type=jax.ShapeDtypeStruct((NC, NP, FEAT), jnp.float32),
        mesh=_mesh,
        scratch_types=[
            pltpu.VMEM((DSCH, 128), jnp.int32),
            pltpu.VMEM((128, FEAT), jnp.float32),
            pltpu.VMEM_SHARED((NP, FEAT), jnp.float32),
            pltpu.SemaphoreType.DMA,
        ],
    )


def _make_agg_kernel():
    return pl.kernel(
        _agg_body,
        out_type=jax.ShapeDtypeStruct((NC, NP, FEAT), jnp.float32),
        mesh=_mesh,
        scratch_types=[
            pltpu.VMEM((SCH, 128), jnp.int32),
            pltpu.VMEM((SCH, 128), jnp.int32),
            pltpu.VMEM((2, CHUNK, FEAT), jnp.float32),
            pltpu.VMEM_SHARED((NP, FEAT), jnp.float32),
            pltpu.SemaphoreType.DMA,
            pltpu.SemaphoreType.DMA,
        ],
    )


def _tc1_body(x_ref, w_ref, dm_ref, g_ref, dinv_ref):
    deg = 1.0 + dm_ref[0, :, 0:1] + dm_ref[1, :, 0:1]
    dinv = jnp.broadcast_to(lax.rsqrt(deg), (RB, FEAT))
    h = jnp.dot(x_ref[...], w_ref[...], preferred_element_type=jnp.float32)
    g_ref[...] = h * dinv
    dinv_ref[...] = dinv


def _tc2_body(agg_ref, g_ref, dinv_ref, b_ref, w_ref, out_ref):
    z = dinv_ref[...] * (agg_ref[0] + agg_ref[1] + g_ref[...]) + b_ref[...]
    z = jnp.maximum(z, 0.0)
    out_ref[...] = dinv_ref[...] * jnp.dot(
        z, w_ref[...], preferred_element_type=jnp.float32)


def _tc3_body(agg_ref, g_ref, dinv_ref, b_ref, wl_ref, bl_ref,
              h2_ref, ls_ref):
    h2 = dinv_ref[...] * (agg_ref[0] + agg_ref[1] + g_ref[...]) + b_ref[...]
    logits = jnp.dot(h2, wl_ref[...],
                     preferred_element_type=jnp.float32) + bl_ref[...]
    m = jnp.max(logits, axis=-1, keepdims=True)
    lse = m + jnp.log(jnp.sum(jnp.exp(logits - m), axis=-1, keepdims=True))
    h2_ref[...] = h2
    ls_ref[...] = logits - lse


def _row_spec(width):
    return pl.BlockSpec((RB, width), lambda i: (i, 0))


def _full_spec(shape):
    nd = len(shape)
    return pl.BlockSpec(shape, lambda i: (0,) * nd)


_pair_spec = pl.BlockSpec((NC, RB, FEAT), lambda i: (0, i, 0))
_grid = (NP // RB,)


def _tc1(x_pad, w1, dm):
    return pl.pallas_call(
        _tc1_body,
        grid=_grid,
        in_specs=[_row_spec(FEAT), _full_spec((FEAT, FEAT)),
                  _pair_spec],
        out_specs=[_row_spec(FEAT), _row_spec(FEAT)],
        out_shape=[jax.ShapeDtypeStruct((NP, FEAT), jnp.float32),
                   jax.ShapeDtypeStruct((NP, FEAT), jnp.float32)],
    )(x_pad, w1, dm)


def _tc2(agg, g1, dinvb, b1, w2):
    return pl.pallas_call(
        _tc2_body,
        grid=_grid,
        in_specs=[_pair_spec, _row_spec(FEAT), _row_spec(FEAT),
                  _full_spec((1, FEAT)), _full_spec((FEAT, FEAT))],
        out_specs=_row_spec(FEAT),
        out_shape=jax.ShapeDtypeStruct((NP, FEAT), jnp.float32),
    )(agg, g1, dinvb, b1, w2)


def _tc3(agg, g2, dinvb, b2, wl_pad, bl_pad):
    return pl.pallas_call(
        _tc3_body,
        grid=_grid,
        in_specs=[_pair_spec, _row_spec(FEAT), _row_spec(FEAT),
                  _full_spec((1, FEAT)), _full_spec((FEAT, FEAT)),
                  _full_spec((1, FEAT))],
        out_specs=[_row_spec(FEAT), _row_spec(FEAT)],
        out_shape=[jax.ShapeDtypeStruct((NP, FEAT), jnp.float32),
                   jax.ShapeDtypeStruct((NP, FEAT), jnp.float32)],
    )(agg, g2, dinvb, b2, wl_pad, bl_pad)


@jax.jit
def kernel(x, edge_index, W1, b1, W2, b2, Wl, bl):
    n, f = x.shape
    e = edge_index.shape[1]
    cls = Wl.shape[1]

    quantum = NS * SCH * (SKEW[0] + SKEW[1]) * 128
    quantum = quantum * DSCH * 128 // math.gcd(quantum, DSCH * 128 * NW)
    e_pad = -(-e // quantum) * quantum
    src_p = jnp.concatenate(
        [edge_index[0], jnp.full((e_pad - e,), n, jnp.int32)])
    dst_p = jnp.concatenate(
        [edge_index[1], jnp.full((e_pad - e,), n, jnp.int32)])
    src2d = src_p.reshape(e_pad // 128, 128)
    dst2d = dst_p.reshape(e_pad // 128, 128)

    x_pad = jnp.zeros((NP, f), jnp.float32).at[:n].set(x)
    onesf = jnp.ones((128, FEAT), jnp.float32)
    zerosf = jnp.zeros((RPT, FEAT), jnp.float32)
    wl_pad = jnp.zeros((FEAT, FEAT), jnp.float32).at[:, :cls].set(Wl)
    bl_pad = jnp.full((1, FEAT), -1e30, jnp.float32).at[0, :cls].set(bl)
    b1r = b1.reshape(1, FEAT)
    b2r = b2.reshape(1, FEAT)

    deg_k = _make_deg_kernel()
    agg_k = _make_agg_kernel()

    dm = deg_k(dst2d, onesf, zerosf)
    g1, dinvb = _tc1(x_pad, W1, dm)
    agg1 = agg_k(g1, src2d, dst2d, zerosf)
    g2 = _tc2(agg1, g1, dinvb, b1r, W2)
    agg2 = agg_k(g2, src2d, dst2d, zerosf)
    h2, ls = _tc3(agg2, g2, dinvb, b2r, wl_pad, bl_pad)
    return h2[:n], ls[:n, :cls]


# speedup vs baseline: 1.2632x; 1.2632x over previous
import functools
import math

import jax
import jax.numpy as jnp
from jax import lax
from jax.experimental import pallas as pl
from jax.experimental.pallas import tpu as pltpu
import jax.experimental.pallas.tpu_sc as plsc

N_NODES = 10000
FEAT = 128
NP = 10240
NC = 2
NS = 16
NW = NC * NS
CHUNK = 128
SCH = 16
DSCH = 40
RPT = NP // NS
RB = 1024
SKEW = (8, 2)

_mesh = plsc.VectorSubcoreMesh(core_axis_name="c", subcore_axis_name="s")


def _deg_body(dst2d, ones_hbm, zeros_hbm, out, didx_v, ones_v, acc_sh, sem):
    cid = lax.axis_index("c")
    sid = lax.axis_index("s")
    wid = sid * NC + cid
    rows_pw = dst2d.shape[0] // NW
    base = wid * rows_pw

    pltpu.sync_copy(zeros_hbm, acc_sh.at[pl.ds(sid * RPT, RPT)])
    pltpu.sync_copy(ones_hbm, ones_v)
    plsc.subcore_barrier()

    for s in range(rows_pw // DSCH):
        pltpu.sync_copy(dst2d.at[pl.ds(base + s * DSCH, DSCH)], didx_v)
        for c0 in range(0, DSCH, 8):
            descs = [
                pltpu.async_copy(ones_v, acc_sh.at[didx_v.at[c0 + t]],
                                 sem, add=True)
                for t in range(8)
            ]
            for d in descs:
                d.wait()

    plsc.subcore_barrier()
    pltpu.sync_copy(acc_sh.at[pl.ds(sid * RPT, RPT)],
                    out.at[cid, pl.ds(sid * RPT, RPT)])


def _agg_body(g_hbm, src2d, dst2d, zeros_hbm, out,
              sidx_v, didx_v, rows_v, acc_sh, gs0, gs1):
    cid = lax.axis_index("c")
    sid = lax.axis_index("s")
    gsem = (gs0, gs1)

    pltpu.sync_copy(zeros_hbm, acc_sh.at[pl.ds(sid * RPT, RPT)])
    plsc.subcore_barrier()

    total_sch = src2d.shape[0] // SCH
    per_pair = total_sch // NS
    nsc0 = per_pair * SKEW[0] // (SKEW[0] + SKEW[1])
    nsc1 = per_pair - nsc0
    rows0 = NS * nsc0 * SCH

    def run(base_row, nsc):
        for s in range(nsc):
            pltpu.sync_copy(src2d.at[pl.ds(base_row + s * SCH, SCH)], sidx_v)
            pltpu.sync_copy(dst2d.at[pl.ds(base_row + s * SCH, SCH)], didx_v)
            pend = [
                pltpu.async_copy(g_hbm.at[sidx_v.at[b]], rows_v.at[b],
                                 gsem[b])
                for b in range(2)
            ]
            for c in range(SCH):
                b = c % 2
                pend[b].wait()
                pltpu.sync_copy(rows_v.at[b], acc_sh.at[didx_v.at[c]],
                                add=True)
                if c + 2 < SCH:
                    pend[b] = pltpu.async_copy(
                        g_hbm.at[sidx_v.at[c + 2]], rows_v.at[b], gsem[b])

    @pl.when(cid == 0)
    def _():
        run(sid * nsc0 * SCH, nsc0)

    @pl.when(cid == 1)
    def _():
        run(rows0 + sid * nsc1 * SCH, nsc1)

    plsc.subcore_barrier()
    pltpu.sync_copy(acc_sh.at[pl.ds(sid * RPT, RPT)],
                    out.at[cid, pl.ds(sid * RPT, RPT)])


def _make_deg_kernel():
    return pl.kernel(
        _deg_body,
        out_type=jax.ShapeDtypeStruct((NC, NP, FEAT), jnp.float32),
        mesh=_mesh,
        scratch_types=[
            pltpu.VMEM((DSCH, 128), jnp.int32),
            pltpu.VMEM((128, FEAT), jnp.float32),
            pltpu.VMEM_SHARED((NP, FEAT), jnp.float32),
            pltpu.SemaphoreType.DMA,
        ],
    )


def _make_agg_kernel():
    return pl.kernel(
        _agg_body,
        out_type=jax.ShapeDtypeStruct((NC, NP, FEAT), jnp.float32),
        mesh=_mesh,
        scratch_types=[
            pltpu.VMEM((SCH, 128), jnp.int32),
            pltpu.VMEM((SCH, 128), jnp.int32),
            pltpu.VMEM((2, CHUNK, FEAT), jnp.float32),
            pltpu.VMEM_SHARED((NP, FEAT), jnp.float32),
            pltpu.SemaphoreType.DMA,
            pltpu.SemaphoreType.DMA,
        ],
    )


def _tc1_body(x_ref, w_ref, dm_ref, g_ref, dinv_ref):
    deg = 1.0 + dm_ref[0, :, 0:1] + dm_ref[1, :, 0:1]
    dinv = jnp.broadcast_to(lax.rsqrt(deg), (RB, FEAT))
    h = jnp.dot(x_ref[...], w_ref[...], preferred_element_type=jnp.float32)
    g_ref[...] = h * dinv
    dinv_ref[...] = dinv


def _tc2_body(agg_ref, g_ref, dinv_ref, b_ref, w_ref, out_ref):
    z = dinv_ref[...] * (agg_ref[0] + agg_ref[1] + g_ref[...]) + b_ref[...]
    z = jnp.maximum(z, 0.0)
    out_ref[...] = dinv_ref[...] * jnp.dot(
        z, w_ref[...], preferred_element_type=jnp.float32)


def _tc3_body(agg_ref, g_ref, dinv_ref, b_ref, wl_ref, bl_ref,
              h2_ref, ls_ref):
    h2 = dinv_ref[...] * (agg_ref[0] + agg_ref[1] + g_ref[...]) + b_ref[...]
    logits = jnp.dot(h2, wl_ref[...],
                     preferred_element_type=jnp.float32) + bl_ref[...]
    m = jnp.max(logits, axis=-1, keepdims=True)
    lse = m + jnp.log(jnp.sum(jnp.exp(logits - m), axis=-1, keepdims=True))
    h2_ref[...] = h2
    ls_ref[...] = logits - lse


def _row_spec(width):
    return pl.BlockSpec((RB, width), lambda i: (i, 0))


def _full_spec(shape):
    nd = len(shape)
    return pl.BlockSpec(shape, lambda i: (0,) * nd)


_pair_spec = pl.BlockSpec((NC, RB, FEAT), lambda i: (0, i, 0))
_grid = (NP // RB,)


def _tc1(x_pad, w1, dm):
    return pl.pallas_call(
        _tc1_body,
        grid=_grid,
        in_specs=[_row_spec(FEAT), _full_spec((FEAT, FEAT)),
                  _pair_spec],
        out_specs=[_row_spec(FEAT), _row_spec(FEAT)],
        out_shape=[jax.ShapeDtypeStruct((NP, FEAT), jnp.float32),
                   jax.ShapeDtypeStruct((NP, FEAT), jnp.float32)],
    )(x_pad, w1, dm)


def _tc2(agg, g1, dinvb, b1, w2):
    return pl.pallas_call(
        _tc2_body,
        grid=_grid,
        in_specs=[_pair_spec, _row_spec(FEAT), _row_spec(FEAT),
                  _full_spec((1, FEAT)), _full_spec((FEAT, FEAT))],
        out_specs=_row_spec(FEAT),
        out_shape=jax.ShapeDtypeStruct((NP, FEAT), jnp.float32),
    )(agg, g1, dinvb, b1, w2)


def _tc3(agg, g2, dinvb, b2, wl_pad, bl_pad):
    return pl.pallas_call(
        _tc3_body,
        grid=_grid,
        in_specs=[_pair_spec, _row_spec(FEAT), _row_spec(FEAT),
                  _full_spec((1, FEAT)), _full_spec((FEAT, FEAT)),
                  _full_spec((1, FEAT))],
        out_specs=[_row_spec(FEAT), _row_spec(FEAT)],
        out_shape=[jax.ShapeDtypeStruct((NP, FEAT), jnp.float32),
                   jax.ShapeDtypeStruct((NP, FEAT), jnp.float32)],
    )(agg, g2, dinvb, b2, wl_pad, bl_pad)


@jax.jit
def kernel(x, edge_index, W1, b1, W2, b2, Wl, bl):
    n, f = x.shape
    e = edge_index.shape[1]
    cls = Wl.shape[1]

    quantum = NS * SCH * (SKEW[0] + SKEW[1]) * 128
    quantum = quantum * DSCH * 128 // math.gcd(quantum, DSCH * 128 * NW)
    e_pad = -(-e // quantum) * quantum
    src_p = jnp.concatenate(
        [edge_index[0], jnp.full((e_pad - e,), n, jnp.int32)])
    dst_p = jnp.concatenate(
        [edge_index[1], jnp.full((e_pad - e,), n, jnp.int32)])
    src2d = src_p.reshape(e_pad // 128, 128)
    dst2d = dst_p.reshape(e_pad // 128, 128)

    x_pad = jnp.zeros((NP, f), jnp.float32).at[:n].set(x)
    onesf = jnp.ones((128, FEAT), jnp.float32)
    zerosf = jnp.zeros((RPT, FEAT), jnp.float32)
    wl_pad = jnp.zeros((FEAT, FEAT), jnp.float32).at[:, :cls].set(Wl)
    bl_pad = jnp.full((1, FEAT), -1e30, jnp.float32).at[0, :cls].set(bl)
    b1r = b1.reshape(1, FEAT)
    b2r = b2.reshape(1, FEAT)

    deg_k = _make_deg_kernel()
    agg_k = _make_agg_kernel()

    dm = deg_k(dst2d, onesf, zerosf)
    g1, dinvb = _tc1(x_pad, W1, dm)
    agg1 = agg_k(g1, src2d, dst2d, zerosf)
    g2 = _tc2(agg1, g1, dinvb, b1r, W2)
    agg2 = agg_k(g2, src2d, dst2d, zerosf)
    h2, ls = _tc3(agg2, g2, dinvb, b2r, wl_pad, bl_pad)
    return h2[:n], ls[:n, :cls]


# speedup vs baseline: 1.2960x; 1.0260x over previous
import functools
import math

import jax
import jax.numpy as jnp
from jax import lax
from jax.experimental import pallas as pl
from jax.experimental.pallas import tpu as pltpu
import jax.experimental.pallas.tpu_sc as plsc

N_NODES = 10000
FEAT = 128
NP = 10240
NC = 2
NS = 16
NW = NC * NS
CHUNK = 128
SCH = 16
DSCH = 40
RPT = NP // NS
RB = 1024
SKEW = (9, 1)

_mesh = plsc.VectorSubcoreMesh(core_axis_name="c", subcore_axis_name="s")


def _deg_body(dst2d, ones_hbm, zeros_hbm, out, didx_v, ones_v, acc_sh, sem):
    cid = lax.axis_index("c")
    sid = lax.axis_index("s")
    wid = sid * NC + cid
    rows_pw = dst2d.shape[0] // NW
    base = wid * rows_pw

    pltpu.sync_copy(zeros_hbm, acc_sh.at[pl.ds(sid * RPT, RPT)])
    pltpu.sync_copy(ones_hbm, ones_v)
    plsc.subcore_barrier()

    for s in range(rows_pw // DSCH):
        pltpu.sync_copy(dst2d.at[pl.ds(base + s * DSCH, DSCH)], didx_v)
        for c0 in range(0, DSCH, 8):
            descs = [
                pltpu.async_copy(ones_v, acc_sh.at[didx_v.at[c0 + t]],
                                 sem, add=True)
                for t in range(8)
            ]
            for d in descs:
                d.wait()

    plsc.subcore_barrier()
    pltpu.sync_copy(acc_sh.at[pl.ds(sid * RPT, RPT)],
                    out.at[cid, pl.ds(sid * RPT, RPT)])


def _agg_body(g_hbm, src2d, dst2d, zeros_hbm, out,
              sidx_v, didx_v, rows_v, acc_sh, gs0, gs1):
    cid = lax.axis_index("c")
    sid = lax.axis_index("s")
    gsem = (gs0, gs1)

    pltpu.sync_copy(zeros_hbm, acc_sh.at[pl.ds(sid * RPT, RPT)])
    plsc.subcore_barrier()

    total_sch = src2d.shape[0] // SCH
    per_pair = total_sch // NS
    nsc0 = per_pair * SKEW[0] // (SKEW[0] + SKEW[1])
    nsc1 = per_pair - nsc0
    rows0 = NS * nsc0 * SCH

    def run(base_row, nsc):
        for s in range(nsc):
            pltpu.sync_copy(src2d.at[pl.ds(base_row + s * SCH, SCH)], sidx_v)
            pltpu.sync_copy(dst2d.at[pl.ds(base_row + s * SCH, SCH)], didx_v)
            pend = [
                pltpu.async_copy(g_hbm.at[sidx_v.at[b]], rows_v.at[b],
                                 gsem[b])
                for b in range(2)
            ]
            for c in range(SCH):
                b = c % 2
                pend[b].wait()
                pltpu.sync_copy(rows_v.at[b], acc_sh.at[didx_v.at[c]],
                                add=True)
                if c + 2 < SCH:
                    pend[b] = pltpu.async_copy(
                        g_hbm.at[sidx_v.at[c + 2]], rows_v.at[b], gsem[b])

    @pl.when(cid == 0)
    def _():
        run(sid * nsc0 * SCH, nsc0)

    @pl.when(cid == 1)
    def _():
        run(rows0 + sid * nsc1 * SCH, nsc1)

    plsc.subcore_barrier()
    pltpu.sync_copy(acc_sh.at[pl.ds(sid * RPT, RPT)],
                    out.at[cid, pl.ds(sid * RPT, RPT)])


def _make_deg_kernel():
    return pl.kernel(
        _deg_body,
        out_type=jax.ShapeDtypeStruct((NC, NP, FEAT), jnp.float32),
        mesh=_mesh,
        scratch_types=[
            pltpu.VMEM((DSCH, 128), jnp.int32),
            pltpu.VMEM((128, FEAT), jnp.float32),
            pltpu.VMEM_SHARED((NP, FEAT), jnp.float32),
            pltpu.SemaphoreType.DMA,
        ],
    )


def _make_agg_kernel():
    return pl.kernel(
        _agg_body,
        out_type=jax.ShapeDtypeStruct((NC, NP, FEAT), jnp.float32),
        mesh=_mesh,
        scratch_types=[
            pltpu.VMEM((SCH, 128), jnp.int32),
            pltpu.VMEM((SCH, 128), jnp.int32),
            pltpu.VMEM((2, CHUNK, FEAT), jnp.float32),
            pltpu.VMEM_SHARED((NP, FEAT), jnp.float32),
            pltpu.SemaphoreType.DMA,
            pltpu.SemaphoreType.DMA,
        ],
    )


def _tc1_body(x_ref, w_ref, dm_ref, g_ref, dinv_ref):
    deg = 1.0 + dm_ref[0, :, 0:1] + dm_ref[1, :, 0:1]
    dinv = jnp.broadcast_to(lax.rsqrt(deg), (RB, FEAT))
    h = jnp.dot(x_ref[...], w_ref[...], preferred_element_type=jnp.float32)
    g_ref[...] = h * dinv
    dinv_ref[...] = dinv


def _tc2_body(agg_ref, g_ref, dinv_ref, b_ref, w_ref, out_ref):
    z = dinv_ref[...] * (agg_ref[0] + agg_ref[1] + g_ref[...]) + b_ref[...]
    z = jnp.maximum(z, 0.0)
    out_ref[...] = dinv_ref[...] * jnp.dot(
        z, w_ref[...], preferred_element_type=jnp.float32)


def _tc3_body(agg_ref, g_ref, dinv_ref, b_ref, wl_ref, bl_ref,
              h2_ref, ls_ref):
    h2 = dinv_ref[...] * (agg_ref[0] + agg_ref[1] + g_ref[...]) + b_ref[...]
    logits = jnp.dot(h2, wl_ref[...],
                     preferred_element_type=jnp.float32) + bl_ref[...]
    m = jnp.max(logits, axis=-1, keepdims=True)
    lse = m + jnp.log(jnp.sum(jnp.exp(logits - m), axis=-1, keepdims=True))
    h2_ref[...] = h2
    ls_ref[...] = logits - lse


def _row_spec(width):
    return pl.BlockSpec((RB, width), lambda i: (i, 0))


def _full_spec(shape):
    nd = len(shape)
    return pl.BlockSpec(shape, lambda i: (0,) * nd)


_pair_spec = pl.BlockSpec((NC, RB, FEAT), lambda i: (0, i, 0))
_grid = (NP // RB,)


def _tc1(x_pad, w1, dm):
    return pl.pallas_call(
        _tc1_body,
        grid=_grid,
        in_specs=[_row_spec(FEAT), _full_spec((FEAT, FEAT)),
                  _pair_spec],
        out_specs=[_row_spec(FEAT), _row_spec(FEAT)],
        out_shape=[jax.ShapeDtypeStruct((NP, FEAT), jnp.float32),
                   jax.ShapeDtypeStruct((NP, FEAT), jnp.float32)],
    )(x_pad, w1, dm)


def _tc2(agg, g1, dinvb, b1, w2):
    return pl.pallas_call(
        _tc2_body,
        grid=_grid,
        in_specs=[_pair_spec, _row_spec(FEAT), _row_spec(FEAT),
                  _full_spec((1, FEAT)), _full_spec((FEAT, FEAT))],
        out_specs=_row_spec(FEAT),
        out_shape=jax.ShapeDtypeStruct((NP, FEAT), jnp.float32),
    )(agg, g1, dinvb, b1, w2)


def _tc3(agg, g2, dinvb, b2, wl_pad, bl_pad):
    return pl.pallas_call(
        _tc3_body,
        grid=_grid,
        in_specs=[_pair_spec, _row_spec(FEAT), _row_spec(FEAT),
                  _full_spec((1, FEAT)), _full_spec((FEAT, FEAT)),
                  _full_spec((1, FEAT))],
        out_specs=[_row_spec(FEAT), _row_spec(FEAT)],
        out_shape=[jax.ShapeDtypeStruct((NP, FEAT), jnp.float32),
                   jax.ShapeDtypeStruct((NP, FEAT), jnp.float32)],
    )(agg, g2, dinvb, b2, wl_pad, bl_pad)


@jax.jit
def kernel(x, edge_index, W1, b1, W2, b2, Wl, bl):
    n, f = x.shape
    e = edge_index.shape[1]
    cls = Wl.shape[1]

    quantum = NS * SCH * (SKEW[0] + SKEW[1]) * 128
    quantum = quantum * DSCH * 128 // math.gcd(quantum, DSCH * 128 * NW)
    e_pad = -(-e // quantum) * quantum
    src_p = jnp.concatenate(
        [edge_index[0], jnp.full((e_pad - e,), n, jnp.int32)])
    dst_p = jnp.concatenate(
        [edge_index[1], jnp.full((e_pad - e,), n, jnp.int32)])
    src2d = src_p.reshape(e_pad // 128, 128)
    dst2d = dst_p.reshape(e_pad // 128, 128)

    x_pad = jnp.zeros((NP, f), jnp.float32).at[:n].set(x)
    onesf = jnp.ones((128, FEAT), jnp.float32)
    zerosf = jnp.zeros((RPT, FEAT), jnp.float32)
    wl_pad = jnp.zeros((FEAT, FEAT), jnp.float32).at[:, :cls].set(Wl)
    bl_pad = jnp.full((1, FEAT), -1e30, jnp.float32).at[0, :cls].set(bl)
    b1r = b1.reshape(1, FEAT)
    b2r = b2.reshape(1, FEAT)

    deg_k = _make_deg_kernel()
    agg_k = _make_agg_kernel()

    dm = deg_k(dst2d, onesf, zerosf)
    g1, dinvb = _tc1(x_pad, W1, dm)
    agg1 = agg_k(g1, src2d, dst2d, zerosf)
    g2 = _tc2(agg1, g1, dinvb, b1r, W2)
    agg2 = agg_k(g2, src2d, dst2d, zerosf)
    h2, ls = _tc3(agg2, g2, dinvb, b2r, wl_pad, bl_pad)
    return h2[:n], ls[:n, :cls]


# speedup vs baseline: 1.3094x; 1.0103x over previous
import functools
import math

import jax
import jax.numpy as jnp
from jax import lax
from jax.experimental import pallas as pl
from jax.experimental.pallas import tpu as pltpu
import jax.experimental.pallas.tpu_sc as plsc

N_NODES = 10000
FEAT = 128
NP = 10240
NC = 2
NS = 16
NW = NC * NS
CHUNK = 128
SCH = 16
DSCH = 40
RPT = NP // NS
RB = 1024
SKEW = (9, 1)

_mesh = plsc.VectorSubcoreMesh(core_axis_name="c", subcore_axis_name="s")


def _deg_body(dst2d, ones_hbm, zeros_hbm, out, didx_v, ones_v, acc_sh, sem):
    cid = lax.axis_index("c")
    sid = lax.axis_index("s")
    wid = sid * NC + cid
    rows_pw = dst2d.shape[0] // NW
    base = wid * rows_pw

    pltpu.sync_copy(zeros_hbm.at[cid], acc_sh.at[pl.ds(sid * RPT, RPT)])
    pltpu.sync_copy(ones_hbm, ones_v)
    plsc.subcore_barrier()

    for s in range(rows_pw // DSCH):
        pltpu.sync_copy(dst2d.at[pl.ds(base + s * DSCH, DSCH)], didx_v)
        for c0 in range(0, DSCH, 8):
            descs = [
                pltpu.async_copy(ones_v, acc_sh.at[didx_v.at[c0 + t]],
                                 sem, add=True)
                for t in range(8)
            ]
            for d in descs:
                d.wait()

    plsc.subcore_barrier()
    pltpu.sync_copy(acc_sh.at[pl.ds(sid * RPT, RPT)],
                    out.at[cid, pl.ds(sid * RPT, RPT)])


def _agg_body(g_hbm, src2d, dst2d, zeros_hbm, out,
              sidx_v, didx_v, rows_v, acc_sh, gs0, gs1):
    cid = lax.axis_index("c")
    sid = lax.axis_index("s")
    gsem = (gs0, gs1)

    pltpu.sync_copy(zeros_hbm.at[cid], acc_sh.at[pl.ds(sid * RPT, RPT)])
    plsc.subcore_barrier()

    total_sch = src2d.shape[0] // SCH
    per_pair = total_sch // NS
    nsc0 = per_pair * SKEW[0] // (SKEW[0] + SKEW[1])
    nsc1 = per_pair - nsc0
    rows0 = NS * nsc0 * SCH

    def run(base_row, nsc):
        for s in range(nsc):
            pltpu.sync_copy(src2d.at[pl.ds(base_row + s * SCH, SCH)], sidx_v)
            pltpu.sync_copy(dst2d.at[pl.ds(base_row + s * SCH, SCH)], didx_v)
            pend = [
                pltpu.async_copy(g_hbm.at[sidx_v.at[b]], rows_v.at[b],
                                 gsem[b])
                for b in range(2)
            ]
            for c in range(SCH):
                b = c % 2
                pend[b].wait()
                pltpu.sync_copy(rows_v.at[b], acc_sh.at[didx_v.at[c]],
                                add=True)
                if c + 2 < SCH:
                    pend[b] = pltpu.async_copy(
                        g_hbm.at[sidx_v.at[c + 2]], rows_v.at[b], gsem[b])

    @pl.when(cid == 0)
    def _():
        run(sid * nsc0 * SCH, nsc0)

    @pl.when(cid == 1)
    def _():
        run(rows0 + sid * nsc1 * SCH, nsc1)

    plsc.subcore_barrier()
    pltpu.sync_copy(acc_sh.at[pl.ds(sid * RPT, RPT)],
                    out.at[cid, pl.ds(sid * RPT, RPT)])


def _make_deg_kernel():
    return pl.kernel(
        _deg_body,
        out_type=jax.ShapeDtypeStruct((NC, NP, FEAT), jnp.float32),
        mesh=_mesh,
        scratch_types=[
            pltpu.VMEM((DSCH, 128), jnp.int32),
            pltpu.VMEM((128, FEAT), jnp.float32),
            pltpu.VMEM_SHARED((NP, FEAT), jnp.float32),
            pltpu.SemaphoreType.DMA,
        ],
    )


def _make_agg_kernel():
    return pl.kernel(
        _agg_body,
        out_type=jax.ShapeDtypeStruct((NC, NP, FEAT), jnp.float32),
        mesh=_mesh,
        scratch_types=[
            pltpu.VMEM((SCH, 128), jnp.int32),
            pltpu.VMEM((SCH, 128), jnp.int32),
            pltpu.VMEM((2, CHUNK, FEAT), jnp.float32),
            pltpu.VMEM_SHARED((NP, FEAT), jnp.float32),
            pltpu.SemaphoreType.DMA,
            pltpu.SemaphoreType.DMA,
        ],
    )


def _tc1_body(x_ref, w_ref, dm_ref, g_ref, dinv_ref):
    deg = 1.0 + dm_ref[0, :, 0:1] + dm_ref[1, :, 0:1]
    dinv = jnp.broadcast_to(lax.rsqrt(deg), (RB, FEAT))
    h = jnp.dot(x_ref[...], w_ref[...], preferred_element_type=jnp.float32)
    g_ref[...] = h * dinv
    dinv_ref[...] = dinv


def _tc2_body(agg_ref, g_ref, dinv_ref, b_ref, w_ref, out_ref):
    z = dinv_ref[...] * (agg_ref[0] + agg_ref[1] + g_ref[...]) + b_ref[...]
    z = jnp.maximum(z, 0.0)
    out_ref[...] = dinv_ref[...] * jnp.dot(
        z, w_ref[...], preferred_element_type=jnp.float32)


def _tc3_body(agg_ref, g_ref, dinv_ref, b_ref, wl_ref, bl_ref,
              h2_ref, ls_ref):
    h2 = dinv_ref[...] * (agg_ref[0] + agg_ref[1] + g_ref[...]) + b_ref[...]
    logits = jnp.dot(h2, wl_ref[...],
                     preferred_element_type=jnp.float32) + bl_ref[...]
    m = jnp.max(logits, axis=-1, keepdims=True)
    lse = m + jnp.log(jnp.sum(jnp.exp(logits - m), axis=-1, keepdims=True))
    h2_ref[...] = h2
    ls_ref[...] = logits - lse


def _row_spec(width):
    return pl.BlockSpec((RB, width), lambda i: (i, 0))


def _full_spec(shape):
    nd = len(shape)
    return pl.BlockSpec(shape, lambda i: (0,) * nd)


_pair_spec = pl.BlockSpec((NC, RB, FEAT), lambda i: (0, i, 0))
_grid = (NP // RB,)


def _tc1(x_pad, w1, dm):
    return pl.pallas_call(
        _tc1_body,
        grid=_grid,
        in_specs=[_row_spec(FEAT), _full_spec((FEAT, FEAT)),
                  _pair_spec],
        out_specs=[_row_spec(FEAT), _row_spec(FEAT)],
        out_shape=[jax.ShapeDtypeStruct((NP, FEAT), jnp.float32),
                   jax.ShapeDtypeStruct((NP, FEAT), jnp.float32)],
    )(x_pad, w1, dm)


def _tc2(agg, g1, dinvb, b1, w2):
    return pl.pallas_call(
        _tc2_body,
        grid=_grid,
        in_specs=[_pair_spec, _row_spec(FEAT), _row_spec(FEAT),
                  _full_spec((1, FEAT)), _full_spec((FEAT, FEAT))],
        out_specs=_row_spec(FEAT),
        out_shape=jax.ShapeDtypeStruct((NP, FEAT), jnp.float32),
    )(agg, g1, dinvb, b1, w2)


def _tc3(agg, g2, dinvb, b2, wl_pad, bl_pad):
    return pl.pallas_call(
        _tc3_body,
        grid=_grid,
        in_specs=[_pair_spec, _row_spec(FEAT), _row_spec(FEAT),
                  _full_spec((1, FEAT)), _full_spec((FEAT, FEAT)),
                  _full_spec((1, FEAT))],
        out_specs=[_row_spec(FEAT), _row_spec(FEAT)],
        out_shape=[jax.ShapeDtypeStruct((NP, FEAT), jnp.float32),
                   jax.ShapeDtypeStruct((NP, FEAT), jnp.float32)],
    )(agg, g2, dinvb, b2, wl_pad, bl_pad)


@jax.jit
def kernel(x, edge_index, W1, b1, W2, b2, Wl, bl):
    n, f = x.shape
    e = edge_index.shape[1]
    cls = Wl.shape[1]

    quantum = NS * SCH * (SKEW[0] + SKEW[1]) * 128
    quantum = quantum * DSCH * 128 // math.gcd(quantum, DSCH * 128 * NW)
    e_pad = -(-e // quantum) * quantum
    src_p = jnp.concatenate(
        [edge_index[0], jnp.full((e_pad - e,), n, jnp.int32)])
    dst_p = jnp.concatenate(
        [edge_index[1], jnp.full((e_pad - e,), n, jnp.int32)])
    src2d = src_p.reshape(e_pad // 128, 128)
    dst2d = dst_p.reshape(e_pad // 128, 128)

    x_pad = jnp.zeros((NP, f), jnp.float32).at[:n].set(x)
    onesf = jnp.ones((128, FEAT), jnp.float32)
    zerosf = jnp.zeros((NC, RPT, FEAT), jnp.float32)
    wl_pad = jnp.zeros((FEAT, FEAT), jnp.float32).at[:, :cls].set(Wl)
    bl_pad = jnp.full((1, FEAT), -1e30, jnp.float32).at[0, :cls].set(bl)
    b1r = b1.reshape(1, FEAT)
    b2r = b2.reshape(1, FEAT)

    deg_k = _make_deg_kernel()
    agg_k = _make_agg_kernel()

    dm = deg_k(dst2d, onesf, zerosf)
    g1, dinvb = _tc1(x_pad, W1, dm)
    agg1 = agg_k(g1, src2d, dst2d, zerosf)
    g2 = _tc2(agg1, g1, dinvb, b1r, W2)
    agg2 = agg_k(g2, src2d, dst2d, zerosf)
    h2, ls = _tc3(agg2, g2, dinvb, b2r, wl_pad, bl_pad)
    return h2[:n], ls[:n, :cls]
